# Initial kernel scaffold; baseline (speedup 1.0000x reference)
import jax
import jax.numpy as jnp
from jax.experimental import pallas as pl


def kernel(x, edge_index):
    raise NotImplementedError("write your pallas kernel here")


# speedup vs baseline: 10.3852x; 10.3852x over previous
import dataclasses
import functools

import jax
import jax.numpy as jnp
from jax import lax
from jax.experimental import pallas as pl
from jax.experimental.pallas import tpu as pltpu
from jax.experimental.pallas import tpu_sc as plsc

N_NODES = 10000
N_EDGES = 320000
D_FEAT = 128

NC = 2
NS = 16
NW = NC * NS
L = 16

EPW = N_EDGES // NW
BLK = 125
NB = EPW // BLK
N_PAD = 10240
ROWS_PER_SUB = N_PAD // NS
ZCHUNK = 128

_vector_mesh = plsc.VectorSubcoreMesh(core_axis_name="c", subcore_axis_name="s")

_sc_params = pltpu.CompilerParams()
if "needs_layout_passes" in pltpu.CompilerParams.__dataclass_fields__:
    _sc_params = dataclasses.replace(_sc_params, needs_layout_passes=False)


@functools.partial(
    pl.kernel,
    out_type=[
        jax.ShapeDtypeStruct((NW, N_NODES), jnp.float32),
        jax.ShapeDtypeStruct((NW, N_NODES), jnp.float32),
    ],
    mesh=_vector_mesh,
    scratch_types=[
        pltpu.VMEM((EPW,), jnp.int32),
        pltpu.VMEM((N_NODES,), jnp.float32),
    ],
    compiler_params=_sc_params,
)
def _hist_kernel(src_hbm, dst_hbm, srch_hbm, dsth_hbm, idx_v, hist_v):
    c = lax.axis_index("c")
    s = lax.axis_index("s")
    wid = c * NS + s
    ones = jnp.ones((L,), jnp.float32)
    zeros = jnp.zeros((L,), jnp.float32)
    for in_hbm, out_hbm in ((src_hbm, srch_hbm), (dst_hbm, dsth_hbm)):
        pltpu.sync_copy(in_hbm.at[wid], idx_v)

        @pl.loop(0, N_NODES // L)
        def _(i):
            hist_v[pl.ds(i * L, L)] = zeros

        @pl.loop(0, EPW // L)
        def _(i):
            idx = idx_v[pl.ds(i * L, L)]
            plsc.addupdate_scatter(hist_v, [idx], ones)

        pltpu.sync_copy(hist_v, out_hbm.at[wid])


def _scale_body(x_ref, srch_ref, dsth_ref, h_ref, nd_ref):
    out_deg = jnp.maximum(jnp.sum(srch_ref[...], axis=0), 1.0)
    h_ref[...] = x_ref[...] * lax.rsqrt(out_deg)[:, None]
    in_deg = jnp.maximum(jnp.sum(dsth_ref[...], axis=0), 1.0)
    nd_ref[...] = lax.rsqrt(in_deg)[None, :]


_scale_kernel = pl.pallas_call(
    _scale_body,
    out_shape=[
        jax.ShapeDtypeStruct((N_NODES, D_FEAT), jnp.float32),
        jax.ShapeDtypeStruct((1, N_NODES), jnp.float32),
    ],
)


@functools.partial(
    pl.kernel,
    out_type=jax.ShapeDtypeStruct((NC, N_PAD, D_FEAT), jnp.float32),
    mesh=_vector_mesh,
    scratch_types=[
        pltpu.VMEM((NB, BLK), jnp.int32),
        pltpu.VMEM((NB, BLK), jnp.int32),
        pltpu.VMEM((ZCHUNK, D_FEAT), jnp.float32),
        pltpu.VMEM_SHARED((N_PAD, D_FEAT), jnp.float32),
        pltpu.SemaphoreType.DMA,
    ],
    compiler_params=_sc_params,
)
def _scatter_kernel(h_hbm, src_hbm, dst_hbm, out_hbm, src_v, dst_v, rows_v,
                    agg_sh, sem):
    c = lax.axis_index("c")
    s = lax.axis_index("s")
    wid = c * NS + s
    zeros = jnp.zeros((L,), jnp.float32)

    @pl.loop(0, ZCHUNK)
    def _(r):
        @pl.loop(0, D_FEAT // L)
        def _(k):
            rows_v[r, pl.ds(k * L, L)] = zeros

    @pl.loop(0, ROWS_PER_SUB // ZCHUNK)
    def _(k):
        pltpu.sync_copy(
            rows_v, agg_sh.at[pl.ds(s * ROWS_PER_SUB + k * ZCHUNK, ZCHUNK)])

    pltpu.sync_copy(src_hbm.at[wid], src_v)
    pltpu.sync_copy(dst_hbm.at[wid], dst_v)
    plsc.subcore_barrier()

    @pl.loop(0, NB)
    def _(j):
        blk = rows_v.at[pl.ds(0, BLK)]
        pltpu.async_copy(h_hbm.at[src_v.at[j]], blk, sem).wait()
        pltpu.sync_copy(blk, agg_sh.at[dst_v.at[j]], add=True)

    plsc.subcore_barrier()

    pltpu.sync_copy(
        agg_sh.at[pl.ds(s * ROWS_PER_SUB, ROWS_PER_SUB)],
        out_hbm.at[c, pl.ds(s * ROWS_PER_SUB, ROWS_PER_SUB)])


def _combine_body(aggp_ref, nd_ref, out_ref):
    agg = aggp_ref[0, :N_NODES, :] + aggp_ref[1, :N_NODES, :]
    out_ref[...] = agg * nd_ref[0][:, None]


_combine_kernel = pl.pallas_call(
    _combine_body,
    out_shape=jax.ShapeDtypeStruct((N_NODES, D_FEAT), jnp.float32),
)


def kernel(x, edge_index):
    src = edge_index[0].reshape(NW, EPW)
    dst = edge_index[1].reshape(NW, EPW)
    dst_b = edge_index[1].reshape(NW, NB, BLK)
    srch, dsth = _hist_kernel(src, dst)
    h, nd = _scale_kernel(x, srch, dsth)
    src_b = edge_index[0].reshape(NW, NB, BLK)
    aggp = _scatter_kernel(h, src_b, dst_b)
    return _combine_kernel(aggp, nd)


# speedup vs baseline: 13.8468x; 1.3333x over previous
import dataclasses
import functools

import jax
import jax.numpy as jnp
from jax import lax
from jax.experimental import pallas as pl
from jax.experimental.pallas import tpu as pltpu
from jax.experimental.pallas import tpu_sc as plsc

N_NODES = 10000
N_EDGES = 320000
D_FEAT = 128

NC = 2
NS = 16
NW = NC * NS
L = 16

EPW = N_EDGES // NW
BLK = 80
NB = EPW // BLK

_vector_mesh = plsc.VectorSubcoreMesh(core_axis_name="c", subcore_axis_name="s")

_sc_params = pltpu.CompilerParams()
if "needs_layout_passes" in pltpu.CompilerParams.__dataclass_fields__:
    _sc_params = dataclasses.replace(_sc_params, needs_layout_passes=False)


@functools.partial(
    pl.kernel,
    out_type=[
        jax.ShapeDtypeStruct((NW, N_NODES), jnp.float32),
        jax.ShapeDtypeStruct((NW, N_NODES), jnp.float32),
    ],
    mesh=_vector_mesh,
    scratch_types=[
        pltpu.VMEM((EPW,), jnp.int32),
        pltpu.VMEM((N_NODES,), jnp.float32),
    ],
    compiler_params=_sc_params,
)
def _hist_kernel(src_hbm, dst_hbm, srch_hbm, dsth_hbm, idx_v, hist_v):
    c = lax.axis_index("c")
    s = lax.axis_index("s")
    wid = c * NS + s
    ones = jnp.ones((L,), jnp.float32)
    zeros = jnp.zeros((L,), jnp.float32)
    for in_hbm, out_hbm in ((src_hbm, srch_hbm), (dst_hbm, dsth_hbm)):
        pltpu.sync_copy(in_hbm.at[wid], idx_v)

        @pl.loop(0, N_NODES // L)
        def _(i):
            hist_v[pl.ds(i * L, L)] = zeros

        @pl.loop(0, EPW // L)
        def _(i):
            idx = idx_v[pl.ds(i * L, L)]
            plsc.addupdate_scatter(hist_v, [idx], ones)

        pltpu.sync_copy(hist_v, out_hbm.at[wid])


def _scale_body(x_ref, srch_ref, dsth_ref, h_ref, nd_ref):
    out_deg = jnp.maximum(jnp.sum(srch_ref[...], axis=0), 1.0)
    h_ref[...] = x_ref[...] * lax.rsqrt(out_deg)[:, None]
    in_deg = jnp.maximum(jnp.sum(dsth_ref[...], axis=0), 1.0)
    nd_ref[...] = lax.rsqrt(in_deg)[None, :]


_scale_kernel = pl.pallas_call(
    _scale_body,
    out_shape=[
        jax.ShapeDtypeStruct((N_NODES, D_FEAT), jnp.float32),
        jax.ShapeDtypeStruct((1, N_NODES), jnp.float32),
    ],
)


@functools.partial(
    pl.kernel,
    out_type=jax.ShapeDtypeStruct((NC, N_NODES, D_FEAT), jnp.float32),
    mesh=_vector_mesh,
    scratch_types=[
        pltpu.VMEM((EPW,), jnp.int32),
        pltpu.VMEM((NB, BLK), jnp.int32),
        pltpu.VMEM((2 * BLK, D_FEAT), jnp.float32),
        pltpu.VMEM_SHARED((N_NODES, D_FEAT), jnp.float32),
        pltpu.SemaphoreType.DMA,
        pltpu.SemaphoreType.DMA,
    ],
    compiler_params=_sc_params,
)
def _scatter_kernel(h_hbm, src_hbm, dst_hbm, out_hbm, src_v, dst_v, ring_v,
                    agg_sh, sem0, sem1):
    c = lax.axis_index("c")
    s = lax.axis_index("s")
    wid = c * NS + s
    zeros = jnp.zeros((L,), jnp.float32)

    src_cp = pltpu.make_async_copy(src_hbm.at[wid], src_v, sem0)
    dst_cp = pltpu.make_async_copy(dst_hbm.at[wid], dst_v, sem0)
    src_cp.start()
    dst_cp.start()

    @pl.loop(0, 2 * BLK)
    def _(r):
        @pl.loop(0, D_FEAT // L)
        def _(k):
            ring_v[r, pl.ds(k * L, L)] = zeros

    zsrc = ring_v.at[pl.ds(0, BLK)]

    @pl.loop(0, (NB + NS - 1) // NS)
    def _(m):
        chunk = s + m * NS

        @pl.when(chunk < NB)
        def _():
            pltpu.sync_copy(zsrc, agg_sh.at[pl.ds(chunk * BLK, BLK)])

    src_cp.wait()
    dst_cp.wait()
    plsc.subcore_barrier()

    rows0_v = ring_v.at[pl.ds(0, BLK)]
    rows1_v = ring_v.at[pl.ds(BLK, BLK)]
    pltpu.make_async_copy(
        h_hbm.at[src_v.at[pl.ds(0, BLK)]], rows0_v, sem0).start()

    @pl.loop(0, NB // 2)
    def _(t):
        j0 = 2 * t
        pltpu.make_async_copy(
            h_hbm.at[src_v.at[pl.ds((j0 + 1) * BLK, BLK)]], rows1_v,
            sem1).start()
        pltpu.make_async_copy(
            h_hbm.at[src_v.at[pl.ds(j0 * BLK, BLK)]], rows0_v, sem0).wait()
        pltpu.sync_copy(rows0_v, agg_sh.at[dst_v.at[j0]], add=True)
        pltpu.make_async_copy(
            h_hbm.at[src_v.at[pl.ds((j0 + 2) * BLK, BLK)]], rows0_v,
            sem0).start()
        pltpu.make_async_copy(
            h_hbm.at[src_v.at[pl.ds((j0 + 1) * BLK, BLK)]], rows1_v,
            sem1).wait()
        pltpu.sync_copy(rows1_v, agg_sh.at[dst_v.at[j0 + 1]], add=True)

    pltpu.make_async_copy(
        h_hbm.at[src_v.at[pl.ds((NB - 1) * BLK, BLK)]], rows0_v, sem0).wait()
    pltpu.sync_copy(rows0_v, agg_sh.at[dst_v.at[NB - 1]], add=True)

    plsc.subcore_barrier()

    @pl.loop(0, (NB + NS - 1) // NS)
    def _(m):
        chunk = s + m * NS

        @pl.when(chunk < NB)
        def _():
            pltpu.sync_copy(agg_sh.at[pl.ds(chunk * BLK, BLK)],
                            out_hbm.at[c, pl.ds(chunk * BLK, BLK)])


def _combine_body(aggp_ref, nd_ref, out_ref):
    agg = aggp_ref[0] + aggp_ref[1]
    out_ref[...] = agg * nd_ref[0][:, None]


_combine_kernel = pl.pallas_call(
    _combine_body,
    out_shape=jax.ShapeDtypeStruct((N_NODES, D_FEAT), jnp.float32),
)


def kernel(x, edge_index):
    src = edge_index[0].reshape(NW, EPW)
    dst = edge_index[1].reshape(NW, EPW)
    dst_b = edge_index[1].reshape(NW, NB, BLK)
    srch, dsth = _hist_kernel(src, dst)
    h, nd = _scale_kernel(x, srch, dsth)
    aggp = _scatter_kernel(h, src, dst_b)
    return _combine_kernel(aggp, nd)


# speedup vs baseline: 14.1391x; 1.0211x over previous
import dataclasses
import functools

import jax
import jax.numpy as jnp
from jax import lax
from jax.experimental import pallas as pl
from jax.experimental.pallas import tpu as pltpu
from jax.experimental.pallas import tpu_sc as plsc

N_NODES = 10000
N_EDGES = 320000
D_FEAT = 128

NC = 2
NS = 16
NW = NC * NS
L = 16

EPW = N_EDGES // NW
BLK = 80
NB = EPW // BLK

_vector_mesh = plsc.VectorSubcoreMesh(core_axis_name="c", subcore_axis_name="s")

_sc_params = pltpu.CompilerParams()
if "needs_layout_passes" in pltpu.CompilerParams.__dataclass_fields__:
    _sc_params = dataclasses.replace(_sc_params, needs_layout_passes=False)


@functools.partial(
    pl.kernel,
    out_type=[
        jax.ShapeDtypeStruct((NW, N_NODES), jnp.float32),
        jax.ShapeDtypeStruct((NW, N_NODES), jnp.float32),
    ],
    mesh=_vector_mesh,
    scratch_types=[
        pltpu.VMEM((EPW,), jnp.int32),
        pltpu.VMEM((N_NODES,), jnp.float32),
    ],
    compiler_params=_sc_params,
)
def _hist_kernel(src_hbm, dst_hbm, srch_hbm, dsth_hbm, idx_v, hist_v):
    c = lax.axis_index("c")
    s = lax.axis_index("s")
    wid = c * NS + s
    ones = jnp.ones((L,), jnp.float32)
    zeros = jnp.zeros((L,), jnp.float32)
    for in_hbm, out_hbm in ((src_hbm, srch_hbm), (dst_hbm, dsth_hbm)):
        pltpu.sync_copy(in_hbm.at[wid], idx_v)

        @pl.loop(0, N_NODES // L)
        def _(i):
            hist_v[pl.ds(i * L, L)] = zeros

        @pl.loop(0, EPW // L)
        def _(i):
            idx = idx_v[pl.ds(i * L, L)]
            plsc.addupdate_scatter(hist_v, [idx], ones)

        pltpu.sync_copy(hist_v, out_hbm.at[wid])


def _scale_body(x_ref, srch_ref, dsth_ref, h_ref, nd_ref):
    out_deg = jnp.maximum(jnp.sum(srch_ref[...], axis=0), 1.0)
    h_ref[...] = x_ref[...] * lax.rsqrt(out_deg)[:, None]
    in_deg = jnp.maximum(jnp.sum(dsth_ref[...], axis=0), 1.0)
    nd_ref[...] = lax.rsqrt(in_deg)[None, :]


_scale_kernel = pl.pallas_call(
    _scale_body,
    out_shape=[
        jax.ShapeDtypeStruct((N_NODES, D_FEAT), jnp.float32),
        jax.ShapeDtypeStruct((1, N_NODES), jnp.float32),
    ],
)


@functools.partial(
    pl.kernel,
    out_type=jax.ShapeDtypeStruct((NC, N_NODES, D_FEAT), jnp.float32),
    mesh=_vector_mesh,
    scratch_types=[
        pltpu.VMEM((EPW,), jnp.int32),
        pltpu.VMEM((NB, BLK), jnp.int32),
        pltpu.VMEM((2 * BLK, D_FEAT), jnp.float32),
        pltpu.VMEM_SHARED((N_NODES, D_FEAT), jnp.float32),
        pltpu.SemaphoreType.DMA,
        pltpu.SemaphoreType.DMA,
        pltpu.SemaphoreType.DMA,
        pltpu.SemaphoreType.DMA,
    ],
    compiler_params=_sc_params,
)
def _scatter_kernel(h_hbm, src_hbm, dst_hbm, out_hbm, src_v, dst_v, ring_v,
                    agg_sh, sem0, sem1, sem2, sem3):
    c = lax.axis_index("c")
    s = lax.axis_index("s")
    wid = c * NS + s
    zeros = jnp.zeros((L,), jnp.float32)

    src_cp = pltpu.make_async_copy(src_hbm.at[wid], src_v, sem0)
    dst_cp = pltpu.make_async_copy(dst_hbm.at[wid], dst_v, sem0)
    src_cp.start()
    dst_cp.start()

    @pl.loop(0, 2 * BLK)
    def _(r):
        @pl.loop(0, D_FEAT // L)
        def _(k):
            ring_v[r, pl.ds(k * L, L)] = zeros

    zsrc = ring_v.at[pl.ds(0, BLK)]

    @pl.loop(0, (NB + NS - 1) // NS)
    def _(m):
        chunk = s + m * NS

        @pl.when(chunk < NB)
        def _():
            pltpu.sync_copy(zsrc, agg_sh.at[pl.ds(chunk * BLK, BLK)])

    src_cp.wait()
    dst_cp.wait()
    plsc.subcore_barrier()

    GB = BLK // 2
    q0 = ring_v.at[pl.ds(0, GB)]
    q1 = ring_v.at[pl.ds(GB, GB)]
    q2 = ring_v.at[pl.ds(2 * GB, GB)]
    q3 = ring_v.at[pl.ds(3 * GB, GB)]
    lo = ring_v.at[pl.ds(0, BLK)]
    hi = ring_v.at[pl.ds(BLK, BLK)]

    def g_cp(g, q, sem):
        return pltpu.make_async_copy(
            h_hbm.at[src_v.at[pl.ds(g * GB, GB)]], q, sem)

    g_cp(0, q0, sem0).start()
    g_cp(1, q1, sem1).start()
    g_cp(2, q2, sem2).start()
    g_cp(3, q3, sem3).start()

    @pl.loop(0, NB // 2)
    def _(t):
        j0 = 2 * t
        g0 = 4 * t
        g_cp(g0, q0, sem0).wait()
        g_cp(g0 + 1, q1, sem1).wait()
        pltpu.sync_copy(lo, agg_sh.at[dst_v.at[j0]], add=True)
        g_cp(g0 + 4, q0, sem0).start()
        g_cp(g0 + 5, q1, sem1).start()
        g_cp(g0 + 2, q2, sem2).wait()
        g_cp(g0 + 3, q3, sem3).wait()
        pltpu.sync_copy(hi, agg_sh.at[dst_v.at[j0 + 1]], add=True)

        @pl.when(t < NB // 2 - 1)
        def _():
            g_cp(g0 + 6, q2, sem2).start()
            g_cp(g0 + 7, q3, sem3).start()

    g_cp(2 * (NB - 1), q0, sem0).wait()
    g_cp(2 * (NB - 1) + 1, q1, sem1).wait()
    pltpu.sync_copy(lo, agg_sh.at[dst_v.at[NB - 1]], add=True)

    plsc.subcore_barrier()

    @pl.loop(0, (NB + NS - 1) // NS)
    def _(m):
        chunk = s + m * NS

        @pl.when(chunk < NB)
        def _():
            pltpu.sync_copy(agg_sh.at[pl.ds(chunk * BLK, BLK)],
                            out_hbm.at[c, pl.ds(chunk * BLK, BLK)])


def _combine_body(aggp_ref, nd_ref, out_ref):
    agg = aggp_ref[0] + aggp_ref[1]
    out_ref[...] = agg * nd_ref[0][:, None]


_combine_kernel = pl.pallas_call(
    _combine_body,
    out_shape=jax.ShapeDtypeStruct((N_NODES, D_FEAT), jnp.float32),
)


def kernel(x, edge_index):
    src = edge_index[0].reshape(NW, EPW)
    dst = edge_index[1].reshape(NW, EPW)
    dst_b = edge_index[1].reshape(NW, NB, BLK)
    srch, dsth = _hist_kernel(src, dst)
    h, nd = _scale_kernel(x, srch, dsth)
    aggp = _scatter_kernel(h, src, dst_b)
    return _combine_kernel(aggp, nd)


# speedup vs baseline: 14.5607x; 1.0298x over previous
import dataclasses
import functools

import jax
import jax.numpy as jnp
from jax import lax
from jax.experimental import pallas as pl
from jax.experimental.pallas import tpu as pltpu
from jax.experimental.pallas import tpu_sc as plsc

N_NODES = 10000
N_EDGES = 320000
D_FEAT = 128

NC = 2
NS = 16
NW = NC * NS
L = 16

EPW = N_EDGES // NW
BLK = 80
NB = EPW // BLK

_vector_mesh = plsc.VectorSubcoreMesh(core_axis_name="c", subcore_axis_name="s")

_sc_params = pltpu.CompilerParams()
if "needs_layout_passes" in pltpu.CompilerParams.__dataclass_fields__:
    _sc_params = dataclasses.replace(_sc_params, needs_layout_passes=False)


@functools.partial(
    pl.kernel,
    out_type=[
        jax.ShapeDtypeStruct((NW, N_NODES), jnp.float32),
        jax.ShapeDtypeStruct((NW, N_NODES), jnp.float32),
    ],
    mesh=_vector_mesh,
    scratch_types=[
        pltpu.VMEM((EPW,), jnp.int32),
        pltpu.VMEM((N_NODES,), jnp.float32),
    ],
    compiler_params=_sc_params,
)
def _hist_kernel(src_hbm, dst_hbm, srch_hbm, dsth_hbm, idx_v, hist_v):
    c = lax.axis_index("c")
    s = lax.axis_index("s")
    wid = c * NS + s
    ones = jnp.ones((L,), jnp.float32)
    zeros = jnp.zeros((L,), jnp.float32)
    for in_hbm, out_hbm in ((src_hbm, srch_hbm), (dst_hbm, dsth_hbm)):
        pltpu.sync_copy(in_hbm.at[wid], idx_v)

        @pl.loop(0, N_NODES // (8 * L))
        def _(i):
            for u in range(8):
                hist_v[pl.ds(i * 8 * L + u * L, L)] = zeros

        @pl.loop(0, N_NODES // L - N_NODES // (8 * L) * 8)
        def _(i):
            hist_v[pl.ds(N_NODES // (8 * L) * 8 * L + i * L, L)] = zeros

        @pl.loop(0, EPW // (8 * L))
        def _(i):
            for u in range(8):
                idx = idx_v[pl.ds(i * 8 * L + u * L, L)]
                plsc.addupdate_scatter(hist_v, [idx], ones)

        @pl.loop(0, EPW // L - EPW // (8 * L) * 8)
        def _(i):
            idx = idx_v[pl.ds(EPW // (8 * L) * 8 * L + i * L, L)]
            plsc.addupdate_scatter(hist_v, [idx], ones)

        pltpu.sync_copy(hist_v, out_hbm.at[wid])


def _scale_body(x_ref, srch_ref, dsth_ref, h_ref, nd_ref):
    out_deg = jnp.maximum(jnp.sum(srch_ref[...], axis=0), 1.0)
    h_ref[...] = x_ref[...] * lax.rsqrt(out_deg)[:, None]
    in_deg = jnp.maximum(jnp.sum(dsth_ref[...], axis=0), 1.0)
    nd_ref[...] = lax.rsqrt(in_deg)[None, :]


_scale_kernel = pl.pallas_call(
    _scale_body,
    out_shape=[
        jax.ShapeDtypeStruct((N_NODES, D_FEAT), jnp.float32),
        jax.ShapeDtypeStruct((1, N_NODES), jnp.float32),
    ],
)


@functools.partial(
    pl.kernel,
    out_type=jax.ShapeDtypeStruct((NC, N_NODES, D_FEAT), jnp.float32),
    mesh=_vector_mesh,
    scratch_types=[
        pltpu.VMEM((EPW,), jnp.int32),
        pltpu.VMEM((NB, BLK), jnp.int32),
        pltpu.VMEM((2 * BLK, D_FEAT), jnp.float32),
        pltpu.VMEM_SHARED((N_NODES, D_FEAT), jnp.float32),
        pltpu.SemaphoreType.DMA,
        pltpu.SemaphoreType.DMA,
        pltpu.SemaphoreType.DMA,
        pltpu.SemaphoreType.DMA,
    ],
    compiler_params=_sc_params,
)
def _scatter_kernel(h_hbm, src_hbm, dst_hbm, out_hbm, src_v, dst_v, ring_v,
                    agg_sh, sem0, sem1, sem2, sem3):
    c = lax.axis_index("c")
    s = lax.axis_index("s")
    wid = c * NS + s
    zeros = jnp.zeros((L,), jnp.float32)

    src_cp = pltpu.make_async_copy(src_hbm.at[wid], src_v, sem0)
    dst_cp = pltpu.make_async_copy(dst_hbm.at[wid], dst_v, sem0)
    src_cp.start()
    dst_cp.start()

    @pl.loop(0, 2 * BLK)
    def _(r):
        @pl.loop(0, D_FEAT // L)
        def _(k):
            ring_v[r, pl.ds(k * L, L)] = zeros

    zsrc = ring_v.at[pl.ds(0, BLK)]

    @pl.loop(0, (NB + NS - 1) // NS)
    def _(m):
        chunk = s + m * NS

        @pl.when(chunk < NB)
        def _():
            pltpu.sync_copy(zsrc, agg_sh.at[pl.ds(chunk * BLK, BLK)])

    src_cp.wait()
    dst_cp.wait()
    plsc.subcore_barrier()

    GB = BLK // 2
    q0 = ring_v.at[pl.ds(0, GB)]
    q1 = ring_v.at[pl.ds(GB, GB)]
    q2 = ring_v.at[pl.ds(2 * GB, GB)]
    q3 = ring_v.at[pl.ds(3 * GB, GB)]
    lo = ring_v.at[pl.ds(0, BLK)]
    hi = ring_v.at[pl.ds(BLK, BLK)]

    def g_cp(g, q, sem):
        return pltpu.make_async_copy(
            h_hbm.at[src_v.at[pl.ds(g * GB, GB)]], q, sem)

    g_cp(0, q0, sem0).start()
    g_cp(1, q1, sem1).start()
    g_cp(2, q2, sem2).start()
    g_cp(3, q3, sem3).start()

    @pl.loop(0, NB // 2)
    def _(t):
        j0 = 2 * t
        g0 = 4 * t
        g_cp(g0, q0, sem0).wait()
        g_cp(g0 + 1, q1, sem1).wait()
        pltpu.sync_copy(lo, agg_sh.at[dst_v.at[j0]], add=True)
        g_cp(g0 + 4, q0, sem0).start()
        g_cp(g0 + 5, q1, sem1).start()
        g_cp(g0 + 2, q2, sem2).wait()
        g_cp(g0 + 3, q3, sem3).wait()
        pltpu.sync_copy(hi, agg_sh.at[dst_v.at[j0 + 1]], add=True)

        @pl.when(t < NB // 2 - 1)
        def _():
            g_cp(g0 + 6, q2, sem2).start()
            g_cp(g0 + 7, q3, sem3).start()

    g_cp(2 * (NB - 1), q0, sem0).wait()
    g_cp(2 * (NB - 1) + 1, q1, sem1).wait()
    pltpu.sync_copy(lo, agg_sh.at[dst_v.at[NB - 1]], add=True)

    plsc.subcore_barrier()

    @pl.loop(0, (NB + NS - 1) // NS)
    def _(m):
        chunk = s + m * NS

        @pl.when(chunk < NB)
        def _():
            pltpu.sync_copy(agg_sh.at[pl.ds(chunk * BLK, BLK)],
                            out_hbm.at[c, pl.ds(chunk * BLK, BLK)])


def _combine_body(aggp_ref, nd_ref, out_ref):
    agg = aggp_ref[0] + aggp_ref[1]
    out_ref[...] = agg * nd_ref[0][:, None]


_combine_kernel = pl.pallas_call(
    _combine_body,
    out_shape=jax.ShapeDtypeStruct((N_NODES, D_FEAT), jnp.float32),
)


def kernel(x, edge_index):
    src = edge_index[0].reshape(NW, EPW)
    dst = edge_index[1].reshape(NW, EPW)
    dst_b = edge_index[1].reshape(NW, NB, BLK)
    srch, dsth = _hist_kernel(src, dst)
    h, nd = _scale_kernel(x, srch, dsth)
    aggp = _scatter_kernel(h, src, dst_b)
    return _combine_kernel(aggp, nd)
